# baseline (speedup 1.0000x reference)
import math

import jax
import jax.numpy as jnp
from jax.experimental import pallas as pl

_N = 1024
_H = 16
_K = 8
_R_MAX = 5.0
_EPS = 1e-8
_AVG_NB = float(_N - 1)
_T = 2

_S0 = 3.1415925994720157
_S1 = -5.1677080818450705
_S2 = 2.5500509887600358
_S3 = -0.5981614666896089
_S4 = 0.07744687538918765
_LOG2E = 1.4426950408889634
_DMU = _R_MAX / (_K - 1)
_RHO = [float(math.exp(-(_DMU * k) ** 2)) for k in range(_K)]


def _mace_body(pos_ref, posT_ref, nfT_ref, seT_ref, wrbf_ref, wrbfT_ref,
               wupdT_ref, wvec_ref, gf_ref, wglob_ref, fs_ref, out_ref):
    pos = pos_ref[:]
    posT = posT_ref[:]

    onehotT = (jax.lax.broadcasted_iota(jnp.int32, (8, _N), 0)
               == nfT_ref[:]).astype(jnp.float32)
    hT = jnp.dot(seT_ref[:], onehotT,
                 preferred_element_type=jnp.float32)

    gram = jnp.dot(pos, posT, preferred_element_type=jnp.float32)
    sq_c = jnp.sum(pos * pos, axis=1, keepdims=True)
    sq_r = jnp.sum(posT * posT, axis=0, keepdims=True)
    d2 = jnp.maximum((sq_c - 2.0 * gram) + sq_r, 0.0) + _EPS
    d2c = jnp.minimum(d2, _R_MAX * _R_MAX)
    lc = jnp.sqrt(d2c)
    inv_len = jax.lax.rsqrt(d2)

    y = lc * (1.0 / _R_MAX) - 0.5
    z = y * y
    sin_pi_y = y * (_S0 + z * (_S1 + z * (_S2 + z * (_S3 + z * _S4))))
    cut = jnp.where(d2 < 1e-4, 0.0, 0.5 - 0.5 * sin_pi_y)

    base = jnp.exp2(-d2c * _LOG2E) * cut
    u = jnp.exp2((2.0 * _DMU * _LOG2E) * lc)

    vec = jnp.zeros((_N, 3), jnp.float32)
    for t in range(_T):
        wrbf = wrbf_ref[t]
        wrbfT = wrbfT_ref[t]
        wvec = wvec_ref[t]
        m_kc = wrbf * wvec
        qT = jnp.dot(m_kc, hT, preferred_element_type=jnp.float32)

        aggT = jnp.zeros((_H, _N), jnp.float32)
        tacc = jnp.zeros((_N, _N), jnp.float32)
        rbf = base
        for k in range(_K):
            aggT = aggT + jnp.dot(hT, rbf,
                                  preferred_element_type=jnp.float32) * (
                                      wrbfT[:, k:k + 1] * _RHO[k])
            tacc = tacc + rbf * (qT[k:k + 1, :] * _RHO[k])
            if k + 1 < _K:
                rbf = rbf * u

        aggT = aggT * (1.0 / _AVG_NB)
        hT = jnp.tanh(jnp.dot(wupdT_ref[t], aggT,
                              preferred_element_type=jnp.float32)) + hT

        tmat = tacc * inv_len
        rowsum = jnp.sum(tmat, axis=1, keepdims=True)
        tp = jnp.dot(tmat, pos, preferred_element_type=jnp.float32)
        vec = vec + (rowsum * pos - tp) * (1.0 / _AVG_NB)

    gate = 1.0 + jnp.tanh(jnp.sum(gf_ref[:] * wglob_ref[:]))
    out_ref[:] = (vec * gate - pos) * fs_ref[0, 0]


def kernel(positions, node_features, global_features, species_embed,
           W_rbf, W_upd, w_vec, w_glob, final_scaling):
    pos = positions.astype(jnp.float32)
    posT = pos.T
    nfT = node_features.astype(jnp.int32).reshape(1, _N)
    seT = jnp.zeros((8, _H), jnp.float32).at[:species_embed.shape[0]].set(
        species_embed.astype(jnp.float32)).T
    wrbf = W_rbf.astype(jnp.float32)
    wrbfT = jnp.swapaxes(wrbf, 1, 2)
    wupdT = jnp.swapaxes(W_upd.astype(jnp.float32), 1, 2)
    wvec = w_vec.astype(jnp.float32).reshape(_T, 1, _H)
    gf = global_features.astype(jnp.float32).reshape(1, -1)
    wglob = w_glob.astype(jnp.float32).reshape(1, -1)
    fs = final_scaling.astype(jnp.float32).reshape(1, 1)

    out = pl.pallas_call(
        _mace_body,
        out_shape=jax.ShapeDtypeStruct((_N, 3), jnp.float32),
    )(pos, posT, nfT, seT, wrbf, wrbfT, wupdT, wvec, gf, wglob, fs)
    return out


# speedup vs baseline: 1.0989x; 1.0989x over previous
import math

import jax
import jax.numpy as jnp
from jax.experimental import pallas as pl

_N = 1024
_H = 16
_K = 8
_NS = 5
_R_MAX = 5.0
_EPS = 1e-8
_AVG_NB = float(_N - 1)
_T = 2

_S0 = 3.1415925994720157
_S1 = -5.1677080818450705
_S2 = 2.5500509887600358
_S3 = -0.5981614666896089
_S4 = 0.07744687538918765
_LOG2E = 1.4426950408889634
_DMU = _R_MAX / (_K - 1)
_RHO = [float(math.exp(-(_DMU * k) ** 2)) for k in range(_K)]


def _mace_body(pos_ref, nfT_ref, se_ref, wrbf_ref, wupd_ref,
               wvec_ref, gf_ref, wglob_ref, fs_ref, out_ref):
    pos = pos_ref[:]
    posT = jnp.transpose(pos)

    onehotT = (jax.lax.broadcasted_iota(jnp.int32, (_NS, _N), 0)
               == nfT_ref[:]).astype(jnp.float32)
    hT = jnp.dot(jnp.transpose(se_ref[:]), onehotT,
                 preferred_element_type=jnp.float32)

    gram = jnp.dot(pos, posT, preferred_element_type=jnp.float32)
    sq_c = jnp.sum(pos * pos, axis=1, keepdims=True)
    sq_r = jnp.sum(posT * posT, axis=0, keepdims=True)
    d2 = jnp.maximum((sq_c - 2.0 * gram) + sq_r, 0.0) + _EPS
    d2c = jnp.minimum(d2, _R_MAX * _R_MAX)
    lc = jnp.sqrt(d2c)
    inv_len = jax.lax.rsqrt(d2)

    y = lc * (1.0 / _R_MAX) - 0.5
    z = y * y
    sin_pi_y = y * (_S0 + z * (_S1 + z * (_S2 + z * (_S3 + z * _S4))))
    cut = jnp.where(d2 < 1e-4, 0.0, 0.5 - 0.5 * sin_pi_y)

    base = jnp.exp2(-d2c * _LOG2E) * cut
    u = jnp.exp2((2.0 * _DMU * _LOG2E) * lc)

    vec = jnp.zeros((_N, 3), jnp.float32)
    for t in range(_T):
        wrbf = wrbf_ref[t]
        wrbfT = jnp.transpose(wrbf)
        wvec = wvec_ref[t]
        m_kc = wrbf * wvec
        qT = jnp.dot(m_kc, hT, preferred_element_type=jnp.float32)

        aggT = jnp.zeros((_H, _N), jnp.float32)
        tacc = jnp.zeros((_N, _N), jnp.float32)
        rbf = base
        for k in range(_K):
            aggT = aggT + jnp.dot(hT, rbf,
                                  preferred_element_type=jnp.float32) * (
                                      wrbfT[:, k:k + 1] * _RHO[k])
            tacc = tacc + rbf * (qT[k:k + 1, :] * _RHO[k])
            if k + 1 < _K:
                rbf = rbf * u

        aggT = aggT * (1.0 / _AVG_NB)
        hT = jnp.tanh(jnp.dot(jnp.transpose(wupd_ref[t]), aggT,
                              preferred_element_type=jnp.float32)) + hT

        tmat = tacc * inv_len
        rowsum = jnp.sum(tmat, axis=1, keepdims=True)
        tp = jnp.dot(tmat, pos, preferred_element_type=jnp.float32)
        vec = vec + (rowsum * pos - tp) * (1.0 / _AVG_NB)

    gate = 1.0 + jnp.tanh(jnp.sum(gf_ref[:] * wglob_ref[:]))
    out_ref[:] = (vec * gate - pos) * fs_ref[0, 0]


def kernel(positions, node_features, global_features, species_embed,
           W_rbf, W_upd, w_vec, w_glob, final_scaling):
    pos = positions.astype(jnp.float32)
    nfT = node_features.astype(jnp.int32).reshape(1, _N)
    se = species_embed.astype(jnp.float32)
    wrbf = W_rbf.astype(jnp.float32)
    wupd = W_upd.astype(jnp.float32)
    wvec = w_vec.astype(jnp.float32).reshape(_T, 1, _H)
    gf = global_features.astype(jnp.float32).reshape(1, -1)
    wglob = w_glob.astype(jnp.float32).reshape(1, -1)
    fs = final_scaling.astype(jnp.float32).reshape(1, 1)

    out = pl.pallas_call(
        _mace_body,
        out_shape=jax.ShapeDtypeStruct((_N, 3), jnp.float32),
    )(pos, nfT, se, wrbf, wupd, wvec, gf, wglob, fs)
    return out


# speedup vs baseline: 1.1888x; 1.0818x over previous
import math

import jax
import jax.numpy as jnp
from jax.experimental import pallas as pl

_N = 1024
_H = 16
_K = 8
_NS = 5
_R_MAX = 5.0
_EPS = 1e-8
_AVG_NB = float(_N - 1)
_T = 2

_S0 = 3.1415925994720157
_S1 = -5.1677080818450705
_S2 = 2.5500509887600358
_S3 = -0.5981614666896089
_S4 = 0.07744687538918765
_LOG2E = 1.4426950408889634
_DMU = _R_MAX / (_K - 1)
_RHO = [float(math.exp(-(_DMU * k) ** 2)) for k in range(_K)]


def _mace_body(pos_ref, nfT_ref, se_ref, wrbf_ref, wupd_ref,
               wvec_ref, gf_ref, wglob_ref, fs_ref, out_ref):
    pos = pos_ref[:]
    posT = jnp.transpose(pos)

    onehotT = (jax.lax.broadcasted_iota(jnp.int32, (_NS, _N), 0)
               == nfT_ref[:]).astype(jnp.float32)
    hT = jnp.dot(jnp.transpose(se_ref[:]), onehotT,
                 preferred_element_type=jnp.float32)

    gram = jnp.dot(pos, posT, preferred_element_type=jnp.float32)
    sq_c = jnp.sum(pos * pos, axis=1, keepdims=True)
    sq_r = jnp.sum(posT * posT, axis=0, keepdims=True)
    d2 = jnp.maximum((sq_c - 2.0 * gram) + sq_r, 0.0) + _EPS
    d2c = jnp.minimum(d2, _R_MAX * _R_MAX)
    lc = jnp.sqrt(d2c)
    inv_len = jax.lax.rsqrt(d2)

    y = lc * (1.0 / _R_MAX) - 0.5
    z = y * y
    sin_pi_y = y * (_S0 + z * (_S1 + z * (_S2 + z * (_S3 + z * _S4))))
    cut = jnp.where(d2 < 1e-4, 0.0, 0.5 - 0.5 * sin_pi_y)

    base = jnp.exp2(-d2c * _LOG2E) * cut
    u = jnp.exp2((2.0 * _DMU * _LOG2E) * lc)

    vec = jnp.zeros((_N, 3), jnp.float32)
    for t in range(_T):
        wrbf = wrbf_ref[t]
        wrbfT = jnp.transpose(wrbf)
        wvec = wvec_ref[t]
        m_kc = wrbf * wvec
        qT = jnp.dot(m_kc, hT, preferred_element_type=jnp.float32)

        aggT = jnp.zeros((_H, _N), jnp.float32)
        tacc = jnp.zeros((_N, _N), jnp.bfloat16)
        rbf = base
        for k in range(_K):
            aggT = aggT + jnp.dot(hT, rbf,
                                  preferred_element_type=jnp.float32) * (
                                      wrbfT[:, k:k + 1] * _RHO[k])
            tacc = tacc + rbf.astype(jnp.bfloat16) * (
                qT[k:k + 1, :] * _RHO[k]).astype(jnp.bfloat16)
            if k + 1 < _K:
                rbf = rbf * u

        aggT = aggT * (1.0 / _AVG_NB)
        hT = jnp.tanh(jnp.dot(jnp.transpose(wupd_ref[t]), aggT,
                              preferred_element_type=jnp.float32)) + hT

        tmat = tacc.astype(jnp.float32) * inv_len
        rowsum = jnp.sum(tmat, axis=1, keepdims=True)
        tp = jnp.dot(tmat, pos, preferred_element_type=jnp.float32)
        vec = vec + (rowsum * pos - tp) * (1.0 / _AVG_NB)

    gate = 1.0 + jnp.tanh(jnp.sum(gf_ref[:] * wglob_ref[:]))
    out_ref[:] = (vec * gate - pos) * fs_ref[0, 0]


def kernel(positions, node_features, global_features, species_embed,
           W_rbf, W_upd, w_vec, w_glob, final_scaling):
    pos = positions.astype(jnp.float32)
    nfT = node_features.astype(jnp.int32).reshape(1, _N)
    se = species_embed.astype(jnp.float32)
    wrbf = W_rbf.astype(jnp.float32)
    wupd = W_upd.astype(jnp.float32)
    wvec = w_vec.astype(jnp.float32).reshape(_T, 1, _H)
    gf = global_features.astype(jnp.float32).reshape(1, -1)
    wglob = w_glob.astype(jnp.float32).reshape(1, -1)
    fs = final_scaling.astype(jnp.float32).reshape(1, 1)

    out = pl.pallas_call(
        _mace_body,
        out_shape=jax.ShapeDtypeStruct((_N, 3), jnp.float32),
    )(pos, nfT, se, wrbf, wupd, wvec, gf, wglob, fs)
    return out
